# Initial kernel scaffold; baseline (speedup 1.0000x reference)
import jax
import jax.numpy as jnp
from jax.experimental import pallas as pl


def kernel(input, target):
    raise NotImplementedError("write your pallas kernel here")


# speedup vs baseline: 34.9997x; 34.9997x over previous
import jax
import jax.numpy as jnp
from jax import lax
from jax.experimental import pallas as pl
from jax.experimental.pallas import tpu as pltpu
from jax.experimental.pallas import tpu_sc as plsc

_B, _C, _H, _W = 4, 19, 512, 512
_P = _H * _W
_N = 2048
_NC, _NS, _L = 2, 16, 16
_NW = _NC * _NS
_WPI = _NW // _B
_PW = _P // _WPI
_CH = 512
_NCHUNK = _PW // _CH
_NGRP = _CH // _L


def _sc_body(x_hbm, t_hbm, out_hbm, logit_v, lbl_v, ha_v, hf_v):
    cid = lax.axis_index("c")
    sid = lax.axis_index("s")
    wid = sid * _NC + cid
    b = wid // _WPI
    base = (wid % _WPI) * _PW

    zero = jnp.zeros((_L,), jnp.float32)

    def zinit(i, carry):
        ha_v[pl.ds(i * _L, _L)] = zero
        hf_v[pl.ds(i * _L, _L)] = zero
        return carry

    lax.fori_loop(0, _C * _N // _L, zinit, 0)

    ones = jnp.ones((_L,), jnp.float32)
    nf = jnp.float32(_N)

    def chunk_body(ch, carry):
        start = base + ch * _CH
        pltpu.sync_copy(x_hbm.at[b, :, pl.ds(start, _CH)], logit_v)
        pltpu.sync_copy(t_hbm.at[b, pl.ds(start, _CH)], lbl_v)

        def grp(g, gcarry):
            off = g * _L
            xs = [logit_v[c, pl.ds(off, _L)] for c in range(_C)]
            m = xs[0]
            for c in range(1, _C):
                m = jnp.maximum(m, xs[c])
            es = [jnp.exp(xc - m) for xc in xs]
            ssum = es[0]
            for c in range(1, _C):
                ssum = ssum + es[c]
            inv = 1.0 / ssum
            lbl = lbl_v[pl.ds(off, _L)]
            fgbin = jnp.zeros((_L,), jnp.int32)
            for c in range(_C):
                p = es[c] * inv
                fg = lbl == c
                err = jnp.where(fg, 1.0 - p, p)
                bin_ = jnp.minimum((err * nf).astype(jnp.int32), _N - 1)
                plsc.addupdate_scatter(ha_v, [bin_ + c * _N], ones)
                fgbin = jnp.where(fg, bin_, fgbin)
            plsc.addupdate_scatter(hf_v, [lbl * _N + fgbin], ones)
            return gcarry

        lax.fori_loop(0, _NGRP, grp, 0)
        return carry

    lax.fori_loop(0, _NCHUNK, chunk_body, 0)
    pltpu.sync_copy(ha_v, out_hbm.at[wid, 0])
    pltpu.sync_copy(hf_v, out_hbm.at[wid, 1])


def _tc_body(h_ref, o_ref):
    h = h_ref[...]
    h4 = h.reshape(_B, _WPI, 2, _C * _N).sum(axis=1)
    ha = h4[:, 0, :].reshape(_B * _C, _N)
    hf = h4[:, 1, :].reshape(_B * _C, _N)

    rows = _B * _C
    blk = 128
    nblk = _N // blk

    ii = lax.broadcasted_iota(jnp.int32, (blk, blk), 0)
    jj = lax.broadcasted_iota(jnp.int32, (blk, blk), 1)
    tri_incl = (ii <= jj).astype(jnp.float32)
    ii2 = lax.broadcasted_iota(jnp.int32, (nblk, nblk), 0)
    jj2 = lax.broadcasted_iota(jnp.int32, (nblk, nblk), 1)
    tri_excl = (ii2 < jj2).astype(jnp.float32)

    def incl_cumsum(x):
        y = lax.dot_general(x.reshape(rows * nblk, blk), tri_incl,
                            (((1,), (0,)), ((), ())),
                            preferred_element_type=jnp.float32)
        t = x.reshape(rows, nblk, blk).sum(axis=2)
        e = lax.dot_general(t, tri_excl, (((1,), (0,)), ((), ())),
                            preferred_element_type=jnp.float32)
        return (y.reshape(rows, nblk, blk) + e[:, :, None]).reshape(rows, _N)

    ia = incl_cumsum(ha)
    ig = incl_cumsum(hf)
    total = ia[:, _N - 1:_N]
    g = ig[:, _N - 1:_N]
    s = total - ia[:, : _N - 1]
    a = g - ig[:, : _N - 1]
    u = g + s - a
    f = jnp.where(u > 0, s / jnp.maximum(u, 1.0), 0.0)
    loss = (0.5 + jnp.sum(f, axis=1, keepdims=True)) / _N
    present = (g > 0).astype(jnp.float32)
    tot = jnp.sum((loss * present).reshape(_B, _C), axis=1)
    cnt = jnp.sum(present.reshape(_B, _C), axis=1)
    per_img = jnp.where(cnt > 0, tot / jnp.maximum(cnt, 1.0), 0.0)
    o_ref[...] = jnp.mean(per_img)[None, None]


def kernel(input, target):
    x = input.reshape(_B, _C, _P)
    t = target.astype(jnp.int32).reshape(_B, _P)

    mesh = plsc.VectorSubcoreMesh(core_axis_name="c", subcore_axis_name="s")
    sc = pl.kernel(
        _sc_body,
        out_type=jax.ShapeDtypeStruct((_NW, 2, _C * _N), jnp.float32),
        mesh=mesh,
        scratch_types=[
            pltpu.VMEM((_C, _CH), jnp.float32),
            pltpu.VMEM((_CH,), jnp.int32),
            pltpu.VMEM((_C * _N,), jnp.float32),
            pltpu.VMEM((_C * _N,), jnp.float32),
        ],
        compiler_params=pltpu.CompilerParams(needs_layout_passes=False),
    )
    hists = sc(x, t)

    out = pl.pallas_call(
        _tc_body,
        out_shape=jax.ShapeDtypeStruct((1, 1), jnp.float32),
    )(hists)
    return out.reshape(())


# speedup vs baseline: 109.3833x; 3.1253x over previous
import jax
import jax.numpy as jnp
from jax import lax
from jax.experimental import pallas as pl
from jax.experimental.pallas import tpu as pltpu
from jax.experimental.pallas import tpu_sc as plsc

_B, _C, _H, _W = 4, 19, 512, 512
_P = _H * _W
_N = 2048
_NC, _NS, _L = 2, 16, 16
_NW = _NC * _NS
_WPI = _NW // _B
_RW = _H // _WPI
_NGRP = _W // _L


def _row_compute(logit_v, lbl_v, ha_v, hf_v, par):
    ones = jnp.ones((_L,), jnp.float32)
    nf = jnp.float32(_N)

    def grp(g, gcarry):
        off = g * _L
        xs = [logit_v[par, c, pl.ds(off, _L)] for c in range(_C)]
        m = xs[0]
        for c in range(1, _C):
            m = jnp.maximum(m, xs[c])
        es = [jnp.exp(xc - m) for xc in xs]
        ssum = es[0]
        for c in range(1, _C):
            ssum = ssum + es[c]
        ninv = nf / ssum
        lbl = lbl_v[par, pl.ds(off, _L)]
        fgbin = jnp.zeros((_L,), jnp.int32)
        for c in range(_C):
            fg = lbl == c
            sel = jnp.where(fg, ssum - es[c], es[c])
            bin_ = jnp.minimum((sel * ninv).astype(jnp.int32), _N - 1)
            plsc.addupdate_scatter(ha_v, [bin_ + c * _N], ones)
            fgbin = jnp.where(fg, bin_, fgbin)
        plsc.addupdate_scatter(hf_v, [lbl * _N + fgbin], ones)
        return gcarry

    lax.fori_loop(0, _NGRP, grp, 0)


def _sc_body(x_hbm, t_hbm, out_hbm, logit_v, lbl_v, ha_v, hf_v,
             semx0, semx1, semt0, semt1):
    cid = lax.axis_index("c")
    sid = lax.axis_index("s")
    wid = sid * _NC + cid
    b = wid // _WPI
    row0 = (wid % _WPI) * _RW
    semx = (semx0, semx1)
    semt = (semt0, semt1)

    def issue(ch, par):
        r = row0 + ch
        pltpu.async_copy(x_hbm.at[b, :, r, :], logit_v.at[par], semx[par])
        pltpu.async_copy(t_hbm.at[b, r, :], lbl_v.at[par], semt[par])

    def wait(par):
        pltpu.make_async_copy(x_hbm.at[b, :, row0, :], logit_v.at[par],
                              semx[par]).wait()
        pltpu.make_async_copy(t_hbm.at[b, row0, :], lbl_v.at[par],
                              semt[par]).wait()

    issue(0, 0)
    issue(1, 1)

    zero = jnp.zeros((_L,), jnp.float32)

    def zinit(i, carry):
        ha_v[pl.ds(i * _L, _L)] = zero
        hf_v[pl.ds(i * _L, _L)] = zero
        return carry

    lax.fori_loop(0, _C * _N // _L, zinit, 0)

    def chunk_pair(i, carry):
        for par in (0, 1):
            ch = i * 2 + par
            wait(par)
            _row_compute(logit_v, lbl_v, ha_v, hf_v, par)
            issue(ch + 2, par)
        return carry

    lax.fori_loop(0, _RW // 2 - 1, chunk_pair, 0)
    for par in (0, 1):
        wait(par)
        _row_compute(logit_v, lbl_v, ha_v, hf_v, par)

    pltpu.sync_copy(ha_v, out_hbm.at[wid, 0])
    pltpu.sync_copy(hf_v, out_hbm.at[wid, 1])


def _tc_body(h_ref, o_ref):
    h = h_ref[...]
    h4 = h.reshape(_B, _WPI, 2, _C * _N).sum(axis=1)
    ha = h4[:, 0, :].reshape(_B * _C, _N)
    hf = h4[:, 1, :].reshape(_B * _C, _N)

    rows = _B * _C
    blk = 128
    nblk = _N // blk

    ii = lax.broadcasted_iota(jnp.int32, (blk, blk), 0)
    jj = lax.broadcasted_iota(jnp.int32, (blk, blk), 1)
    tri_incl = (ii <= jj).astype(jnp.float32)
    ii2 = lax.broadcasted_iota(jnp.int32, (nblk, nblk), 0)
    jj2 = lax.broadcasted_iota(jnp.int32, (nblk, nblk), 1)
    tri_excl = (ii2 < jj2).astype(jnp.float32)

    def incl_cumsum(x):
        y = lax.dot_general(x.reshape(rows * nblk, blk), tri_incl,
                            (((1,), (0,)), ((), ())),
                            preferred_element_type=jnp.float32)
        t = x.reshape(rows, nblk, blk).sum(axis=2)
        e = lax.dot_general(t, tri_excl, (((1,), (0,)), ((), ())),
                            preferred_element_type=jnp.float32)
        return (y.reshape(rows, nblk, blk) + e[:, :, None]).reshape(rows, _N)

    ia = incl_cumsum(ha)
    ig = incl_cumsum(hf)
    total = ia[:, _N - 1:_N]
    g = ig[:, _N - 1:_N]
    s = total - ia[:, : _N - 1]
    a = g - ig[:, : _N - 1]
    u = g + s - a
    f = jnp.where(u > 0, s / jnp.maximum(u, 1.0), 0.0)
    loss = (0.5 + jnp.sum(f, axis=1, keepdims=True)) / _N
    present = (g > 0).astype(jnp.float32)
    tot = jnp.sum((loss * present).reshape(_B, _C), axis=1)
    cnt = jnp.sum(present.reshape(_B, _C), axis=1)
    per_img = jnp.where(cnt > 0, tot / jnp.maximum(cnt, 1.0), 0.0)
    o_ref[...] = jnp.mean(per_img)[None, None]


def kernel(input, target):
    t = target.astype(jnp.int32)

    mesh = plsc.VectorSubcoreMesh(core_axis_name="c", subcore_axis_name="s")
    sc = pl.kernel(
        _sc_body,
        out_type=jax.ShapeDtypeStruct((_NW, 2, _C * _N), jnp.float32),
        mesh=mesh,
        scratch_types=[
            pltpu.VMEM((2, _C, _W), jnp.float32),
            pltpu.VMEM((2, _W), jnp.int32),
            pltpu.VMEM((_C * _N,), jnp.float32),
            pltpu.VMEM((_C * _N,), jnp.float32),
            pltpu.SemaphoreType.DMA,
            pltpu.SemaphoreType.DMA,
            pltpu.SemaphoreType.DMA,
            pltpu.SemaphoreType.DMA,
        ],
        compiler_params=pltpu.CompilerParams(needs_layout_passes=False),
    )
    hists = sc(input, t)

    out = pl.pallas_call(
        _tc_body,
        out_shape=jax.ShapeDtypeStruct((1, 1), jnp.float32),
    )(hists)
    return out.reshape(())
